# baseline (speedup 1.0000x reference)
import jax
import jax.numpy as jnp
from jax import lax
from jax.experimental import pallas as pl
from jax.experimental.pallas import tpu as pltpu

T, B, D, H, LL = 2048, 8, 512, 512, 4
TT = 256
NT = T // TT


def _body(x_ref, w1t_ref, b1_ref, w0_ref, w2t_ref, w3t_ref, tb_v_ref,
          inv_ref, tb_s_ref, out_ref, x1c_s, sfull_s, g_s, c_s, x2_s,
          w0sum_s, oh_s):
    p = pl.program_id(0)
    i = pl.program_id(1)
    start = i * TT
    t_max = tb_s_ref[LL - 1, 0]
    live = start <= t_max

    @pl.when(p == 0)
    def _phase0():
        @pl.when(i == 0)
        def _init():
            sfull_s[...] = jnp.zeros_like(sfull_s)
            g_s[...] = jnp.zeros_like(g_s)

        @pl.when(live)
        def _stream():
            x = x_ref[...]
            sfull_s[...] = sfull_s[...] + jnp.sum(x, axis=0)
            for j in range(LL):
                for b in range(B):
                    tjb = tb_s_ref[j, b]

                    @pl.when(jnp.logical_and(tjb >= start,
                                             tjb < start + TT))
                    def _g():
                        g_s[pl.ds(j, 1), pl.ds(b, 1), :] = (
                            x_ref[pl.ds(tjb - start, 1), pl.ds(b, 1), :])

            xb = x.astype(jnp.bfloat16).reshape(TT * B, D)
            x1 = jnp.dot(xb, w1t_ref[...],
                         preferred_element_type=jnp.float32) + b1_ref[...]
            x1c_s[pl.ds(start * B, TT * B)] = (0.5 * x1).astype(jnp.bfloat16)

    @pl.when(p == 1)
    def _phase1():
        @pl.when(i == 0)
        def _prologue():
            g = g_s[...]
            s3 = sfull_s[...][None]
            s2 = s3 - g[3][None]
            s1 = s2 - g[2][None]
            s0 = s1 - g[1][None]
            s_all = jnp.concatenate([s0, s1, s2, s3], axis=0)
            g2 = jnp.dot(g.reshape(LL * B, D), w2t_ref[...],
                         preferred_element_type=jnp.float32)
            sw = jnp.dot(s_all.reshape(LL * B, D), w3t_ref[...],
                         preferred_element_type=jnp.float32)
            inv = inv_ref[...].reshape(LL * B, 1)
            x2_s[...] = g2.reshape(LL, B, H)
            c_s[...] = (0.5 * (g2 + sw * inv)).astype(jnp.bfloat16
                                                      ).reshape(LL, B, H)
            w0sum_s[0, 0] = jnp.sum(w0_ref[...])
            rowb = lax.broadcasted_iota(jnp.int32, (TT * B, B), 0) % B
            col = lax.broadcasted_iota(jnp.int32, (TT * B, B), 1)
            oh_s[...] = jnp.where(rowb == col, 1.0, 0.0)
            out_ref[...] = jnp.zeros_like(out_ref)

        @pl.when(live)
        def _attend():
            x1h = x1c_s[pl.ds(start * B, TT * B)]
            x1h3 = x1h.reshape(TT, B, H)
            w0c = w0_ref[...].reshape(H, 1).astype(jnp.bfloat16)
            w0sum = w0sum_s[0, 0]
            oh = oh_s[...]
            tvec = lax.broadcasted_iota(jnp.int32, (TT, B, 1), 0) + start
            tb = tb_v_ref[...]
            for j in range(LL):
                @pl.when(start <= tb_s_ref[j, 0])
                def _att_j():
                    cj = c_s[j][None]
                    th = jnp.tanh(x1h3 + cj)
                    score = w0sum + jnp.dot(th.reshape(TT * B, H), w0c,
                                            preferred_element_type=jnp.float32)
                    le = (tvec <= tb[j][None]).astype(jnp.float32)
                    scorem = (score.reshape(TT, B, 1) * le).reshape(TT * B, 1)
                    mt = (oh * scorem).T.astype(jnp.bfloat16)
                    out_ref[j] = out_ref[j] + jnp.dot(
                        mt, x1h, preferred_element_type=jnp.float32)

        @pl.when(i == NT - 1)
        def _epilogue():
            out_ref[...] = out_ref[...] + x2_s[...]


@jax.jit
def kernel(inputs, lengths, label_len, W1, b1, W2, W3, W0):
    tb_i = lengths[None, :].astype(jnp.int32) - label_len + jnp.arange(LL)[:, None]
    tb_v = tb_i.reshape(LL, B, 1)
    inv = 1.0 / (tb_v.astype(jnp.float32) + 1.0)
    w1t = W1.T.astype(jnp.bfloat16)

    out = pl.pallas_call(
        _body,
        grid=(2, NT),
        in_specs=[
            pl.BlockSpec((TT, B, D), lambda p, i: ((1 - p) * i, 0, 0)),
            pl.BlockSpec((D, H), lambda p, i: (0, 0)),
            pl.BlockSpec((1, H), lambda p, i: (0, 0)),
            pl.BlockSpec((1, H), lambda p, i: (0, 0)),
            pl.BlockSpec((D, H), lambda p, i: (0, 0)),
            pl.BlockSpec((D, H), lambda p, i: (0, 0)),
            pl.BlockSpec((LL, B, 1), lambda p, i: (0, 0, 0)),
            pl.BlockSpec((LL, B, 1), lambda p, i: (0, 0, 0)),
            pl.BlockSpec(memory_space=pltpu.SMEM),
        ],
        out_specs=pl.BlockSpec((LL, B, H), lambda p, i: (0, 0, 0)),
        out_shape=jax.ShapeDtypeStruct((LL, B, H), jnp.float32),
        scratch_shapes=[
            pltpu.VMEM((T * B, H), jnp.bfloat16),
            pltpu.VMEM((B, D), jnp.float32),
            pltpu.VMEM((LL, B, D), jnp.float32),
            pltpu.VMEM((LL, B, H), jnp.bfloat16),
            pltpu.VMEM((LL, B, H), jnp.float32),
            pltpu.SMEM((1, 1), jnp.float32),
            pltpu.VMEM((TT * B, B), jnp.float32),
        ],
    )(inputs, w1t, b1.reshape(1, H), W0.reshape(1, H),
      W2.T, W3.T, tb_v, inv, tb_i)

    return jnp.transpose(out, (1, 0, 2))


# speedup vs baseline: 1.0373x; 1.0373x over previous
import jax
import jax.numpy as jnp
from jax import lax
from jax.experimental import pallas as pl
from jax.experimental.pallas import tpu as pltpu

T, B, D, H, LL = 2048, 8, 512, 512, 4
TT = 256
NT = T // TT


def _body(x_ref, w1t_ref, b1_ref, w0_ref, w2t_ref, w3t_ref, tb_v_ref,
          inv_ref, tb_s_ref, out_ref, xc_s, sfull_s, g_s, c_s, x2_s,
          w0sum_s):
    p = pl.program_id(0)
    i = pl.program_id(1)
    start = i * TT
    t_max = tb_s_ref[LL - 1, 0]
    live = start <= t_max

    @pl.when(p == 0)
    def _phase0():
        @pl.when(i == 0)
        def _init():
            sfull_s[...] = jnp.zeros_like(sfull_s)
            g_s[...] = jnp.zeros_like(g_s)

        @pl.when(live)
        def _accum():
            x = x_ref[...]
            xc_s[pl.ds(start * B, TT * B)] = x.astype(jnp.bfloat16).reshape(TT * B, D)
            sfull_s[...] = sfull_s[...] + jnp.sum(x, axis=0)
            for j in range(LL):
                for b in range(B):
                    tjb = tb_s_ref[j, b]

                    @pl.when(jnp.logical_and(tjb >= start,
                                             tjb < start + TT))
                    def _g():
                        g_s[pl.ds(j, 1), pl.ds(b, 1), :] = (
                            x_ref[pl.ds(tjb - start, 1), pl.ds(b, 1), :])

    @pl.when(p == 1)
    def _phase1():
        @pl.when(i == 0)
        def _prologue():
            g = g_s[...]
            s3 = sfull_s[...][None]
            s2 = s3 - g[3][None]
            s1 = s2 - g[2][None]
            s0 = s1 - g[1][None]
            s_all = jnp.concatenate([s0, s1, s2, s3], axis=0)
            g2 = jnp.dot(g.reshape(LL * B, D), w2t_ref[...],
                         preferred_element_type=jnp.float32)
            sw = jnp.dot(s_all.reshape(LL * B, D), w3t_ref[...],
                         preferred_element_type=jnp.float32)
            inv = inv_ref[...].reshape(LL * B, 1)
            x2_s[...] = g2.reshape(LL, B, H)
            c_s[...] = (0.5 * (g2 + sw * inv)).reshape(LL, B, H)
            w0sum_s[0, 0] = 0.5 * jnp.sum(w0_ref[...])
            out_ref[...] = jnp.zeros_like(out_ref)

        @pl.when(live)
        def _attend():
            xb = xc_s[pl.ds(start * B, TT * B)]
            x1 = jnp.dot(xb, w1t_ref[...],
                         preferred_element_type=jnp.float32) + b1_ref[...]
            x1 = x1.reshape(TT, B, H)
            x1h = 0.5 * x1
            w0c = (0.5 * w0_ref[...]).reshape(H, 1).astype(jnp.bfloat16)
            w0sum = w0sum_s[0, 0]
            tvec = lax.broadcasted_iota(jnp.int32, (TT, B, 1), 0) + start
            tb = tb_v_ref[...]
            for j in range(LL):
                @pl.when(start <= tb_s_ref[j, 0])
                def _att_j():
                    cj = c_s[j][None]
                    th = jnp.tanh(x1h + cj).astype(jnp.bfloat16)
                    score = w0sum + jnp.dot(th.reshape(TT * B, H), w0c,
                                            preferred_element_type=jnp.float32)
                    score = score.reshape(TT, B, 1)
                    le = (tvec <= tb[j][None]).astype(jnp.float32)
                    out_ref[j] = out_ref[j] + jnp.sum(x1 * (score * le),
                                                      axis=0)

        @pl.when(i == NT - 1)
        def _epilogue():
            out_ref[...] = out_ref[...] + x2_s[...]


@jax.jit
def kernel(inputs, lengths, label_len, W1, b1, W2, W3, W0):
    tb_i = lengths[None, :].astype(jnp.int32) - label_len + jnp.arange(LL)[:, None]
    tb_v = tb_i.reshape(LL, B, 1)
    inv = 1.0 / (tb_v.astype(jnp.float32) + 1.0)
    w1t = W1.T.astype(jnp.bfloat16)

    out = pl.pallas_call(
        _body,
        grid=(2, NT),
        in_specs=[
            pl.BlockSpec((TT, B, D), lambda p, i: ((1 - p) * i, 0, 0)),
            pl.BlockSpec((D, H), lambda p, i: (0, 0)),
            pl.BlockSpec((1, H), lambda p, i: (0, 0)),
            pl.BlockSpec((1, H), lambda p, i: (0, 0)),
            pl.BlockSpec((D, H), lambda p, i: (0, 0)),
            pl.BlockSpec((D, H), lambda p, i: (0, 0)),
            pl.BlockSpec((LL, B, 1), lambda p, i: (0, 0, 0)),
            pl.BlockSpec((LL, B, 1), lambda p, i: (0, 0, 0)),
            pl.BlockSpec(memory_space=pltpu.SMEM),
        ],
        out_specs=pl.BlockSpec((LL, B, H), lambda p, i: (0, 0, 0)),
        out_shape=jax.ShapeDtypeStruct((LL, B, H), jnp.float32),
        scratch_shapes=[
            pltpu.VMEM((T * B, D), jnp.bfloat16),
            pltpu.VMEM((B, D), jnp.float32),
            pltpu.VMEM((LL, B, D), jnp.float32),
            pltpu.VMEM((LL, B, H), jnp.float32),
            pltpu.VMEM((LL, B, H), jnp.float32),
            pltpu.SMEM((1, 1), jnp.float32),
        ],
    )(inputs, w1t, b1.reshape(1, H), W0.reshape(1, H),
      W2.T, W3.T, tb_v, inv, tb_i)

    return jnp.transpose(out, (1, 0, 2))
